# baseline (speedup 1.0000x reference)
import jax
import jax.numpy as jnp
from jax.experimental import pallas as pl
from jax.experimental.pallas import tpu as pltpu

_B, _H, _W, _C = 16, 32, 32, 256


def _body(row_ref, col_ref, pose_ref, pemb_ref, m_ref, slab_ref, *sems):
    pv = pose_ref[0:1, :]
    prow = jnp.concatenate([pv, pv], axis=1)
    pemb_ref[...] = jnp.broadcast_to(prow, (_B, 2 * _C))

    csl = col_ref[pl.ds(1, _W), :]
    rsl = row_ref[pl.ds(1, _H), :]
    cpart = jnp.broadcast_to(csl[None, None, :, :], (2, _H, _W, _C))
    rpart = jnp.broadcast_to(rsl[None, :, None, :], (2, _H, _W, _C))
    slab_ref[:, :, :, pl.ds(0, _C)] = cpart
    slab_ref[:, :, :, pl.ds(_C, _C)] = rpart

    copies = [pltpu.make_async_copy(slab_ref, m_ref.at[pl.ds(2 * i, 2)],
                                    sems[i % len(sems)])
              for i in range(_B // 2)]
    for cp in copies:
        cp.start()
    for cp in copies:
        cp.wait()


def kernel(x, row_embed, col_embed, pose_token_embed):
    del x
    p_emb, m4 = pl.pallas_call(
        _body,
        in_specs=[
            pl.BlockSpec(memory_space=pltpu.VMEM),
            pl.BlockSpec(memory_space=pltpu.VMEM),
            pl.BlockSpec(memory_space=pltpu.VMEM),
        ],
        out_specs=[
            pl.BlockSpec(memory_space=pltpu.VMEM),
            pl.BlockSpec(memory_space=pl.MemorySpace.ANY),
        ],
        out_shape=[
            jax.ShapeDtypeStruct((_B, 2 * _C), jnp.float32),
            jax.ShapeDtypeStruct((_B, _H, _W, 2 * _C), jnp.float32),
        ],
        scratch_shapes=[
            pltpu.VMEM((2, _H, _W, 2 * _C), jnp.float32),
            pltpu.SemaphoreType.DMA,
            pltpu.SemaphoreType.DMA,
            pltpu.SemaphoreType.DMA,
            pltpu.SemaphoreType.DMA,
        ],
    )(row_embed, col_embed, pose_token_embed)
    m_emb = jnp.transpose(m4, (0, 3, 1, 2))
    return (p_emb, m_emb)


# speedup vs baseline: 1.0207x; 1.0207x over previous
import jax
import jax.numpy as jnp
from jax.experimental import pallas as pl
from jax.experimental.pallas import tpu as pltpu

_B, _H, _W, _C = 16, 32, 32, 256


def _body(row_ref, col_ref, pose_ref, pemb_ref, m_ref, slab_ref, *sems):
    pv = pose_ref[0:1, :]
    prow = jnp.concatenate([pv, pv], axis=1)
    pemb_ref[...] = jnp.broadcast_to(prow, (_B, 2 * _C))

    csl = col_ref[pl.ds(1, _W), :]
    copies = []
    _YQ = 8
    for q in range(_H // _YQ):
        ys = pl.ds(q * _YQ, _YQ)
        rslq = row_ref[pl.ds(1 + q * _YQ, _YQ), :]
        slab_ref[ys, :, pl.ds(0, _C)] = jnp.broadcast_to(
            csl[None, :, :], (_YQ, _W, _C))
        slab_ref[ys, :, pl.ds(_C, _C)] = jnp.broadcast_to(
            rslq[:, None, :], (_YQ, _W, _C))
        for b in range(_B):
            cp = pltpu.make_async_copy(slab_ref.at[ys], m_ref.at[b, ys],
                                       sems[b % len(sems)])
            cp.start()
            copies.append(cp)
    for cp in copies:
        cp.wait()


def kernel(x, row_embed, col_embed, pose_token_embed):
    del x
    p_emb, m4 = pl.pallas_call(
        _body,
        in_specs=[
            pl.BlockSpec(memory_space=pltpu.VMEM),
            pl.BlockSpec(memory_space=pltpu.VMEM),
            pl.BlockSpec(memory_space=pltpu.VMEM),
        ],
        out_specs=[
            pl.BlockSpec(memory_space=pltpu.VMEM),
            pl.BlockSpec(memory_space=pl.MemorySpace.ANY),
        ],
        out_shape=[
            jax.ShapeDtypeStruct((_B, 2 * _C), jnp.float32),
            jax.ShapeDtypeStruct((_B, _H, _W, 2 * _C), jnp.float32),
        ],
        scratch_shapes=[
            pltpu.VMEM((_H, _W, 2 * _C), jnp.float32),
            pltpu.SemaphoreType.DMA,
            pltpu.SemaphoreType.DMA,
            pltpu.SemaphoreType.DMA,
            pltpu.SemaphoreType.DMA,
        ],
    )(row_embed, col_embed, pose_token_embed)
    m_emb = jnp.transpose(m4, (0, 3, 1, 2))
    return (p_emb, m_emb)


# speedup vs baseline: 1.0234x; 1.0026x over previous
import jax
import jax.numpy as jnp
from jax.experimental import pallas as pl
from jax.experimental.pallas import tpu as pltpu

_B, _H, _W, _C = 16, 32, 32, 256


def _body(row_ref, col_ref, pose_ref, pemb_ref, m_ref, slab_ref, *sems):
    pv = pose_ref[0:1, :]
    prow = jnp.concatenate([pv, pv], axis=1)
    pemb_ref[...] = jnp.broadcast_to(prow, (_B, 2 * _C))

    csl = col_ref[pl.ds(1, _W), :]
    copies = []
    _YQ = 4
    for q in range(_H // _YQ):
        ys = pl.ds(q * _YQ, _YQ)
        rslq = row_ref[pl.ds(1 + q * _YQ, _YQ), :]
        slab_ref[ys, :, pl.ds(0, _C)] = jnp.broadcast_to(
            csl[None, :, :], (_YQ, _W, _C))
        slab_ref[ys, :, pl.ds(_C, _C)] = jnp.broadcast_to(
            rslq[:, None, :], (_YQ, _W, _C))
        for b in range(_B):
            cp = pltpu.make_async_copy(slab_ref.at[ys], m_ref.at[b, ys],
                                       sems[b % len(sems)])
            cp.start()
            copies.append(cp)
    for cp in copies:
        cp.wait()


def kernel(x, row_embed, col_embed, pose_token_embed):
    del x
    p_emb, m4 = pl.pallas_call(
        _body,
        in_specs=[
            pl.BlockSpec(memory_space=pltpu.VMEM),
            pl.BlockSpec(memory_space=pltpu.VMEM),
            pl.BlockSpec(memory_space=pltpu.VMEM),
        ],
        out_specs=[
            pl.BlockSpec(memory_space=pltpu.VMEM),
            pl.BlockSpec(memory_space=pl.MemorySpace.ANY),
        ],
        out_shape=[
            jax.ShapeDtypeStruct((_B, 2 * _C), jnp.float32),
            jax.ShapeDtypeStruct((_B, _H, _W, 2 * _C), jnp.float32),
        ],
        scratch_shapes=[
            pltpu.VMEM((_H, _W, 2 * _C), jnp.float32),
            pltpu.SemaphoreType.DMA,
            pltpu.SemaphoreType.DMA,
            pltpu.SemaphoreType.DMA,
            pltpu.SemaphoreType.DMA,
        ],
    )(row_embed, col_embed, pose_token_embed)
    m_emb = jnp.transpose(m4, (0, 3, 1, 2))
    return (p_emb, m_emb)
